# baseline (speedup 1.0000x reference)
import functools

import jax
import jax.numpy as jnp
from jax import lax
from jax.experimental import pallas as pl
from jax.experimental.pallas import tpu as pltpu
from jax.experimental.pallas import tpu_sc as plsc

NA = 5000
NC = 5000
D = 128
TH = 0.045
NP = 5120
ECAP = 196608
NCORE = 2
NSUB = 16
NW = NCORE * NSUB
ETILE = ECAP // NW
GCHUNK = 64
NGCH = ETILE // GCHUNK
SCHUNK = 128
NSCH = ETILE // SCHUNK
NSLOT = 3
ROWS = NP // NSUB
EB = 512
GB = EB // GCHUNK

_HIGH = jax.lax.Precision.HIGHEST


def _gn(x, w, b):
    m = jnp.mean(x, axis=1, keepdims=True)
    v = jnp.mean((x - m) ** 2, axis=1, keepdims=True)
    return (x - m) * jax.lax.rsqrt(v + 1e-5) * w + b


def _mask_body(actr_ref, cctr_ref, out_ref):
    ax = actr_ref[:, 0:1]
    ay = actr_ref[:, 1:2]
    cx = cctr_ref[0:1, :]
    cy = cctr_ref[1:2, :]
    dx = ax - cx
    dy = ay - cy
    dist = jnp.sqrt(dx * dx + dy * dy)
    out_ref[...] = (dist <= TH).astype(jnp.int8)


def _mask_call(actr2, cctr2):
    return pl.pallas_call(
        _mask_body,
        grid=(NP // 256, NP // 512),
        in_specs=[
            pl.BlockSpec((256, 128), lambda i, j: (i, 0)),
            pl.BlockSpec((8, 512), lambda i, j: (0, j)),
        ],
        out_specs=pl.BlockSpec((256, 512), lambda i, j: (i, j)),
        out_shape=jax.ShapeDtypeStruct((NP, NP), jnp.int8),
    )(actr2, cctr2)


def _pre_body(agts_ref, actr_ref, nodes_ref, nctr_ref, wqt_ref, qgw_ref,
              qgb_ref, w1pt_ref, bt_ref, ct_ref, b1_ref,
              ua0_ref, ua1_ref, vc0_ref, vc1_ref):
    agts = agts_ref[...]
    q = jnp.maximum(
        _gn(jnp.dot(agts, wqt_ref[...], precision=_HIGH),
            qgw_ref[...], qgb_ref[...]), 0.0)
    ua0_ref[...] = jnp.dot(actr_ref[...], w1pt_ref[...], precision=_HIGH)
    ua1_ref[...] = jnp.dot(q, bt_ref[...], precision=_HIGH)
    vc0_ref[...] = b1_ref[...] - jnp.dot(nctr_ref[...], w1pt_ref[...],
                                         precision=_HIGH)
    vc1_ref[...] = jnp.dot(nodes_ref[...], ct_ref[...], precision=_HIGH)


def _pre_call(agts, actr2, nodes_p, nctr2, wqt, qgw, qgb, w1pt, bt, ct, b1):
    full = lambda i: (0, 0)
    row = lambda i: (i, 0)
    return pl.pallas_call(
        _pre_body,
        grid=(NP // 512,),
        in_specs=[
            pl.BlockSpec((512, D), row),
            pl.BlockSpec((512, D), row),
            pl.BlockSpec((512, D), row),
            pl.BlockSpec((512, D), row),
            pl.BlockSpec((D, D), full),
            pl.BlockSpec((1, D), full),
            pl.BlockSpec((1, D), full),
            pl.BlockSpec((D, D), full),
            pl.BlockSpec((D, D), full),
            pl.BlockSpec((D, D), full),
            pl.BlockSpec((1, D), full),
        ],
        out_specs=[pl.BlockSpec((512, D), row)] * 4,
        out_shape=[jax.ShapeDtypeStruct((NP, D), jnp.float32)] * 4,
    )(agts, actr2, nodes_p, nctr2, wqt, qgw, qgb, w1pt, bt, ct, b1)


def _sc_gather(ua0, ua1, vc0, vc1, ei2, ej2):
    mesh = plsc.VectorSubcoreMesh(core_axis_name="c", subcore_axis_name="s")

    @functools.partial(
        pl.kernel,
        mesh=mesh,
        name="scgather",
        out_type=jax.ShapeDtypeStruct((ECAP // GCHUNK, 4, GCHUNK, D),
                                      jnp.float32),
        scratch_types=[
            pltpu.VMEM((NGCH, GCHUNK), jnp.int32),
            pltpu.VMEM((NGCH, GCHUNK), jnp.int32),
            [pltpu.VMEM((4, GCHUNK, D), jnp.float32)] * NSLOT,
            [pltpu.SemaphoreType.DMA] * NSLOT,
        ],
    )
    def k(ua0_hbm, ua1_hbm, vc0_hbm, vc1_hbm, ei_hbm, ej_hbm, g_hbm,
          ia, ic, bufs, sg):
        wid = lax.axis_index("s") * NCORE + lax.axis_index("c")
        rbase = wid * NGCH
        pltpu.sync_copy(ei_hbm.at[pl.ds(rbase, NGCH)], ia)
        pltpu.sync_copy(ej_hbm.at[pl.ds(rbase, NGCH)], ic)

        def fire(c, s):
            pltpu.async_copy(ua0_hbm.at[ia.at[c]], bufs[s].at[0], sg[s])
            pltpu.async_copy(ua1_hbm.at[ia.at[c]], bufs[s].at[1], sg[s])
            pltpu.async_copy(vc0_hbm.at[ic.at[c]], bufs[s].at[2], sg[s])
            pltpu.async_copy(vc1_hbm.at[ic.at[c]], bufs[s].at[3], sg[s])

        def drain(c, s):
            pltpu.make_async_copy(ua0_hbm.at[ia.at[c]], bufs[s].at[0],
                                  sg[s]).wait()
            pltpu.make_async_copy(ua1_hbm.at[ia.at[c]], bufs[s].at[1],
                                  sg[s]).wait()
            pltpu.make_async_copy(vc0_hbm.at[ic.at[c]], bufs[s].at[2],
                                  sg[s]).wait()
            pltpu.make_async_copy(vc1_hbm.at[ic.at[c]], bufs[s].at[3],
                                  sg[s]).wait()

        for s in range(NSLOT - 1):
            fire(s, s)

        def group(jg, carry):
            for b in range(NSLOT):
                c = jg * NSLOT + b
                drain(c, b)
                pltpu.sync_copy(bufs[b], g_hbm.at[rbase + c])
                f = c + (NSLOT - 1)
                b2 = (b + NSLOT - 1) % NSLOT

                @pl.when(f < NGCH)
                def _():
                    fire(f, b2)

            return carry

        lax.fori_loop(0, NGCH // NSLOT, group, 0)

    return k(ua0, ua1, vc0, vc1, ei2, ej2)


def _edge_body(g_ref, w2t_ref, at_ref, c2t_ref, dgw_ref, dgb_ref,
               cgw_ref, cgb_ref, out_ref):
    def part(t):
        return jnp.concatenate([g_ref[i, t] for i in range(GB)], axis=0)

    e1 = jnp.maximum(part(0) + part(2), 0.0)
    z = jnp.dot(e1, w2t_ref[...], precision=_HIGH)
    e2 = jnp.maximum(_gn(z, dgw_ref[...], dgb_ref[...]), 0.0)
    h = jnp.dot(e2, at_ref[...], precision=_HIGH) + part(1) + part(3)
    cc = jnp.maximum(_gn(h, cgw_ref[...], cgb_ref[...]), 0.0)
    out_ref[...] = jnp.dot(cc, c2t_ref[...], precision=_HIGH)


def _edge_call(g, w2t, at, c2t, dgw, dgb, cgw, cgb):
    full2 = lambda i: (0, 0)
    return pl.pallas_call(
        _edge_body,
        grid=(ECAP // EB,),
        in_specs=[
            pl.BlockSpec((GB, 4, GCHUNK, D), lambda i: (i, 0, 0, 0)),
            pl.BlockSpec((D, D), full2),
            pl.BlockSpec((D, D), full2),
            pl.BlockSpec((D, D), full2),
            pl.BlockSpec((1, D), full2),
            pl.BlockSpec((1, D), full2),
            pl.BlockSpec((1, D), full2),
            pl.BlockSpec((1, D), full2),
        ],
        out_specs=pl.BlockSpec((EB, D), lambda i: (i, 0)),
        out_shape=jax.ShapeDtypeStruct((ECAP, D), jnp.float32),
    )(g, w2t, at, c2t, dgw, dgb, cgw, cgb)


def _sc_scatter(oute, sidx2, zrows):
    mesh = plsc.VectorSubcoreMesh(core_axis_name="c", subcore_axis_name="s")

    @functools.partial(
        pl.kernel,
        mesh=mesh,
        name="scscatter",
        out_type=jax.ShapeDtypeStruct((NCORE, NP, D), jnp.float32),
        scratch_types=[
            pltpu.VMEM((NSCH, SCHUNK), jnp.int32),
            [pltpu.VMEM((SCHUNK, D), jnp.float32)] * NSLOT,
            pltpu.VMEM_SHARED((NP, D), jnp.float32),
            [pltpu.SemaphoreType.DMA] * NSLOT,
        ],
    )
    def k(oute_hbm, sidx_hbm, zrows_hbm, acc_hbm, ix, buf, shared, sl):
        cid = lax.axis_index("c")
        sid = lax.axis_index("s")
        wid = sid * NCORE + cid
        pltpu.sync_copy(zrows_hbm, shared.at[pl.ds(sid * ROWS, ROWS)])
        pltpu.sync_copy(sidx_hbm.at[pl.ds(wid * NSCH, NSCH)], ix)
        plsc.subcore_barrier()

        ebase = wid * ETILE
        for s in range(NSLOT):
            pltpu.async_copy(oute_hbm.at[pl.ds(ebase + s * SCHUNK, SCHUNK)],
                             buf[s], sl[s])

        def group(jg, carry):
            for b in range(NSLOT):
                c = jg * NSLOT + b
                pltpu.make_async_copy(
                    oute_hbm.at[pl.ds(ebase + c * SCHUNK, SCHUNK)], buf[b],
                    sl[b]).wait()
                pltpu.sync_copy(buf[b], shared.at[ix.at[c]], add=True)
                f = c + NSLOT

                @pl.when(f < NSCH)
                def _():
                    pltpu.async_copy(
                        oute_hbm.at[pl.ds(ebase + f * SCHUNK, SCHUNK)],
                        buf[b], sl[b])

            return carry

        lax.fori_loop(0, NSCH // NSLOT, group, 0)
        plsc.subcore_barrier()
        pltpu.sync_copy(shared.at[pl.ds(sid * ROWS, ROWS)],
                        acc_hbm.at[cid, pl.ds(sid * ROWS, ROWS)])

    return k(oute, sidx2, zrows)


def _post_body(agts_ref, a0_ref, a1_ref, awt_ref, nw_ref, nb_ref, lwt_ref,
               lgw_ref, lgb_ref, out_ref):
    agts = agts_ref[...]
    a = jnp.dot(agts, awt_ref[...], precision=_HIGH) + a0_ref[...] + a1_ref[...]
    a = jnp.maximum(_gn(a, nw_ref[...], nb_ref[...]), 0.0)
    a = _gn(jnp.dot(a, lwt_ref[...], precision=_HIGH), lgw_ref[...],
            lgb_ref[...])
    out_ref[...] = jnp.maximum(a + agts, 0.0)


def _post_call(agts, a0, a1, awt, nw, nb, lwt, lgw, lgb):
    full = lambda i: (0, 0)
    row = lambda i: (i, 0)
    return pl.pallas_call(
        _post_body,
        grid=(NP // 512,),
        in_specs=[
            pl.BlockSpec((512, D), row),
            pl.BlockSpec((512, D), row),
            pl.BlockSpec((512, D), row),
            pl.BlockSpec((D, D), full),
            pl.BlockSpec((1, D), full),
            pl.BlockSpec((1, D), full),
            pl.BlockSpec((D, D), full),
            pl.BlockSpec((1, D), full),
            pl.BlockSpec((1, D), full),
        ],
        out_specs=pl.BlockSpec((512, D), row),
        out_shape=jax.ShapeDtypeStruct((NP, D), jnp.float32),
    )(agts, a0, a1, awt, nw, nb, lwt, lgw, lgb)


def kernel(actors, actor_idcs, actor_ctrs, nodes, node_idcs, node_ctrs,
           params):
    f32 = jnp.float32
    agt_ctrs = actor_ctrs.reshape(-1, 2)
    ctx_ctrs = node_ctrs.reshape(-1, 2)

    actr2 = jnp.full((NP, D), 9.0, f32).at[:NA, :2].set(agt_ctrs)
    actr2 = actr2.at[:NA, 2:].set(0.0)
    cctr2 = jnp.full((8, NP), 9.0, f32).at[:2, :NC].set(ctx_ctrs.T)
    cctr2 = cctr2.at[2:, :].set(0.0)

    mask = _mask_call(actr2, cctr2)
    flat = jnp.nonzero(mask.reshape(-1), size=ECAP,
                       fill_value=NP * NP)[0].astype(jnp.int32)
    valid = flat < NP * NP
    ei = flat // NP
    ej = flat - ei * NP
    ei2 = jnp.where(valid, ei, 0).reshape(ECAP // GCHUNK, GCHUNK)
    ej2 = jnp.where(valid, ej, 0).reshape(ECAP // GCHUNK, GCHUNK)
    sidx2 = jnp.where(valid, ei, NP - 1).reshape(ECAP // SCHUNK, SCHUNK)

    agts = jnp.zeros((NP, D), f32).at[:NA].set(actors)
    nodes_p = jnp.zeros((NP, D), f32).at[:NC].set(nodes)
    nctr2 = jnp.zeros((NP, D), f32).at[:NC, :2].set(ctx_ctrs)
    zrows = jnp.zeros((ROWS, D), f32)

    for i in range(2):
        p = {k: v[i] for k, v in params.items()}
        w1p = jnp.zeros((D, D), f32).at[:, :2].set(p['dist_w1'])
        ua0, ua1, vc0, vc1 = _pre_call(
            agts, actr2, nodes_p, nctr2,
            p['query_w'].T, p['query_gnw'][None], p['query_gnb'][None],
            w1p.T, p['ctx_w1'][:, D:2 * D].T, p['ctx_w1'][:, 2 * D:].T,
            p['dist_b1'][None])
        g = _sc_gather(ua0, ua1, vc0, vc1, ei2, ej2)
        oute = _edge_call(
            g, p['dist_w2'].T, p['ctx_w1'][:, :D].T, p['ctx_w2'].T,
            p['dist_gnw'][None], p['dist_gnb'][None],
            p['ctx_gnw'][None], p['ctx_gnb'][None])
        acc = _sc_scatter(oute, sidx2, zrows)
        agts = _post_call(
            agts, acc[0], acc[1], p['agt_w'].T, p['norm_w'][None],
            p['norm_b'][None], p['lin_w'].T, p['lin_gnw'][None],
            p['lin_gnb'][None])
    return agts[:NA]


# speedup vs baseline: 1.5606x; 1.5606x over previous
import functools

import jax
import jax.numpy as jnp
from jax import lax
from jax.experimental import pallas as pl
from jax.experimental.pallas import tpu as pltpu
from jax.experimental.pallas import tpu_sc as plsc

NA = 5000
NC = 5000
D = 128
TH = 0.045
NP = 5120
ECAP = 196608
NCORE = 2
NSUB = 16
NW = NCORE * NSUB
ETILE = ECAP // NW
GCHUNK = 128
GTILE = ECAP // NSUB
NGCH = GTILE // GCHUNK
SCHUNK = 128
NSCH = ETILE // SCHUNK
NSLOT = 3
ROWS = NP // NSUB
EB = 512

_HIGH = jax.lax.Precision.HIGHEST


def _gn(x, w, b):
    m = jnp.mean(x, axis=1, keepdims=True)
    v = jnp.mean((x - m) ** 2, axis=1, keepdims=True)
    return (x - m) * jax.lax.rsqrt(v + 1e-5) * w + b


def _mask_body(actr_ref, cctr_ref, out_ref):
    ax = actr_ref[:, 0:1]
    ay = actr_ref[:, 1:2]
    cx = cctr_ref[0:1, :]
    cy = cctr_ref[1:2, :]
    dx = ax - cx
    dy = ay - cy
    dist = jnp.sqrt(dx * dx + dy * dy)
    out_ref[...] = (dist <= TH).astype(jnp.int8)


def _mask_call(actr2, cctr2):
    return pl.pallas_call(
        _mask_body,
        grid=(NP // 256, NP // 512),
        in_specs=[
            pl.BlockSpec((256, 128), lambda i, j: (i, 0)),
            pl.BlockSpec((8, 512), lambda i, j: (0, j)),
        ],
        out_specs=pl.BlockSpec((256, 512), lambda i, j: (i, j)),
        out_shape=jax.ShapeDtypeStruct((NP, NP), jnp.int8),
    )(actr2, cctr2)


def _pre_body(agts_ref, actr_ref, nodes_ref, nctr_ref, wqt_ref, qgw_ref,
              qgb_ref, w1pt_ref, bt_ref, ct_ref, b1_ref,
              ua0_ref, ua1_ref, vc0_ref, vc1_ref):
    agts = agts_ref[...]
    q = jnp.maximum(
        _gn(jnp.dot(agts, wqt_ref[...], precision=_HIGH),
            qgw_ref[...], qgb_ref[...]), 0.0)
    ua0_ref[...] = jnp.dot(actr_ref[...], w1pt_ref[...], precision=_HIGH)
    ua1_ref[...] = jnp.dot(q, bt_ref[...], precision=_HIGH)
    vc0_ref[...] = b1_ref[...] - jnp.dot(nctr_ref[...], w1pt_ref[...],
                                         precision=_HIGH)
    vc1_ref[...] = jnp.dot(nodes_ref[...], ct_ref[...], precision=_HIGH)


def _pre_call(agts, actr2, nodes_p, nctr2, wqt, qgw, qgb, w1pt, bt, ct, b1):
    full = lambda i: (0, 0)
    row = lambda i: (i, 0)
    return pl.pallas_call(
        _pre_body,
        grid=(NP // 512,),
        in_specs=[
            pl.BlockSpec((512, D), row),
            pl.BlockSpec((512, D), row),
            pl.BlockSpec((512, D), row),
            pl.BlockSpec((512, D), row),
            pl.BlockSpec((D, D), full),
            pl.BlockSpec((1, D), full),
            pl.BlockSpec((1, D), full),
            pl.BlockSpec((D, D), full),
            pl.BlockSpec((D, D), full),
            pl.BlockSpec((D, D), full),
            pl.BlockSpec((1, D), full),
        ],
        out_specs=[pl.BlockSpec((512, D), row)] * 4,
        out_shape=[jax.ShapeDtypeStruct((NP, D), jnp.float32)] * 4,
    )(agts, actr2, nodes_p, nctr2, wqt, qgw, qgb, w1pt, bt, ct, b1)


def _sc_gather(ua0, ua1, vc0, vc1, ei2, ej2):
    mesh = plsc.VectorSubcoreMesh(core_axis_name="c", subcore_axis_name="s")

    @functools.partial(
        pl.kernel,
        mesh=mesh,
        name="scgather",
        out_type=jax.ShapeDtypeStruct((4, ECAP, D), jnp.float32),
        scratch_types=[
            pltpu.VMEM((NGCH, GCHUNK), jnp.int32),
            pltpu.VMEM((NGCH, GCHUNK), jnp.int32),
            [pltpu.VMEM((GCHUNK, D), jnp.float32)] * NSLOT,
            pltpu.VMEM_SHARED((NP, D), jnp.float32),
            [pltpu.SemaphoreType.DMA] * NSLOT,
        ],
    )
    def k(ua0_hbm, ua1_hbm, vc0_hbm, vc1_hbm, ei_hbm, ej_hbm, g_hbm,
          ixe, ixj, bufs, tbl, sg):
        cid = lax.axis_index("c")
        sid = lax.axis_index("s")
        rows = pl.ds(sid * ROWS, ROWS)
        pltpu.sync_copy(ei_hbm.at[pl.ds(sid * NGCH, NGCH)], ixe)
        pltpu.sync_copy(ej_hbm.at[pl.ds(sid * NGCH, NGCH)], ixj)
        ebase = sid * GTILE

        for r in range(2):
            src0, src1 = (ua0_hbm, ua1_hbm) if r == 0 else (vc0_hbm, vc1_hbm)
            ix = ixe if r == 0 else ixj
            if r == 1:
                plsc.subcore_barrier()

            @pl.when(cid == 0)
            def _():
                pltpu.sync_copy(src0.at[rows], tbl.at[rows])

            @pl.when(cid == 1)
            def _():
                pltpu.sync_copy(src1.at[rows], tbl.at[rows])

            plsc.subcore_barrier()
            t = 2 * r + cid

            def fire(c, s):
                pltpu.async_copy(tbl.at[ix.at[c]], bufs[s], sg[s])

            def drain(c, s):
                pltpu.make_async_copy(tbl.at[ix.at[c]], bufs[s],
                                      sg[s]).wait()

            for s in range(NSLOT - 1):
                fire(s, s)

            def group(jg, carry):
                for b in range(NSLOT):
                    c = jg * NSLOT + b
                    drain(c, b)
                    pltpu.sync_copy(
                        bufs[b], g_hbm.at[t, pl.ds(ebase + c * GCHUNK,
                                                   GCHUNK)])
                    f = c + (NSLOT - 1)
                    b2 = (b + NSLOT - 1) % NSLOT

                    @pl.when(f < NGCH)
                    def _():
                        fire(f, b2)

                return carry

            lax.fori_loop(0, NGCH // NSLOT, group, 0)

    return k(ua0, ua1, vc0, vc1, ei2, ej2)


def _edge_body(g_ref, w2t_ref, at_ref, c2t_ref, dgw_ref, dgb_ref,
               cgw_ref, cgb_ref, out_ref):
    def part(t):
        return g_ref[t]

    e1 = jnp.maximum(part(0) + part(2), 0.0)
    z = jnp.dot(e1, w2t_ref[...], precision=_HIGH)
    e2 = jnp.maximum(_gn(z, dgw_ref[...], dgb_ref[...]), 0.0)
    h = jnp.dot(e2, at_ref[...], precision=_HIGH) + part(1) + part(3)
    cc = jnp.maximum(_gn(h, cgw_ref[...], cgb_ref[...]), 0.0)
    out_ref[...] = jnp.dot(cc, c2t_ref[...], precision=_HIGH)


def _edge_call(g, w2t, at, c2t, dgw, dgb, cgw, cgb):
    full2 = lambda i: (0, 0)
    return pl.pallas_call(
        _edge_body,
        grid=(ECAP // EB,),
        in_specs=[
            pl.BlockSpec((4, EB, D), lambda i: (0, i, 0)),
            pl.BlockSpec((D, D), full2),
            pl.BlockSpec((D, D), full2),
            pl.BlockSpec((D, D), full2),
            pl.BlockSpec((1, D), full2),
            pl.BlockSpec((1, D), full2),
            pl.BlockSpec((1, D), full2),
            pl.BlockSpec((1, D), full2),
        ],
        out_specs=pl.BlockSpec((EB, D), lambda i: (i, 0)),
        out_shape=jax.ShapeDtypeStruct((ECAP, D), jnp.float32),
    )(g, w2t, at, c2t, dgw, dgb, cgw, cgb)


def _sc_scatter(oute, sidx2, zrows):
    mesh = plsc.VectorSubcoreMesh(core_axis_name="c", subcore_axis_name="s")

    @functools.partial(
        pl.kernel,
        mesh=mesh,
        name="scscatter",
        out_type=jax.ShapeDtypeStruct((NCORE, NP, D), jnp.float32),
        scratch_types=[
            pltpu.VMEM((NSCH, SCHUNK), jnp.int32),
            [pltpu.VMEM((SCHUNK, D), jnp.float32)] * NSLOT,
            pltpu.VMEM_SHARED((NP, D), jnp.float32),
            [pltpu.SemaphoreType.DMA] * NSLOT,
        ],
    )
    def k(oute_hbm, sidx_hbm, zrows_hbm, acc_hbm, ix, buf, shared, sl):
        cid = lax.axis_index("c")
        sid = lax.axis_index("s")
        wid = sid * NCORE + cid
        pltpu.sync_copy(zrows_hbm, shared.at[pl.ds(sid * ROWS, ROWS)])
        pltpu.sync_copy(sidx_hbm.at[pl.ds(wid * NSCH, NSCH)], ix)
        plsc.subcore_barrier()

        ebase = wid * ETILE
        for s in range(NSLOT):
            pltpu.async_copy(oute_hbm.at[pl.ds(ebase + s * SCHUNK, SCHUNK)],
                             buf[s], sl[s])

        def group(jg, carry):
            for b in range(NSLOT):
                c = jg * NSLOT + b
                pltpu.make_async_copy(
                    oute_hbm.at[pl.ds(ebase + c * SCHUNK, SCHUNK)], buf[b],
                    sl[b]).wait()
                pltpu.sync_copy(buf[b], shared.at[ix.at[c]], add=True)
                f = c + NSLOT

                @pl.when(f < NSCH)
                def _():
                    pltpu.async_copy(
                        oute_hbm.at[pl.ds(ebase + f * SCHUNK, SCHUNK)],
                        buf[b], sl[b])

            return carry

        lax.fori_loop(0, NSCH // NSLOT, group, 0)
        plsc.subcore_barrier()
        pltpu.sync_copy(shared.at[pl.ds(sid * ROWS, ROWS)],
                        acc_hbm.at[cid, pl.ds(sid * ROWS, ROWS)])

    return k(oute, sidx2, zrows)


def _post_body(agts_ref, a0_ref, a1_ref, awt_ref, nw_ref, nb_ref, lwt_ref,
               lgw_ref, lgb_ref, out_ref):
    agts = agts_ref[...]
    a = jnp.dot(agts, awt_ref[...], precision=_HIGH) + a0_ref[...] + a1_ref[...]
    a = jnp.maximum(_gn(a, nw_ref[...], nb_ref[...]), 0.0)
    a = _gn(jnp.dot(a, lwt_ref[...], precision=_HIGH), lgw_ref[...],
            lgb_ref[...])
    out_ref[...] = jnp.maximum(a + agts, 0.0)


def _post_call(agts, a0, a1, awt, nw, nb, lwt, lgw, lgb):
    full = lambda i: (0, 0)
    row = lambda i: (i, 0)
    return pl.pallas_call(
        _post_body,
        grid=(NP // 512,),
        in_specs=[
            pl.BlockSpec((512, D), row),
            pl.BlockSpec((512, D), row),
            pl.BlockSpec((512, D), row),
            pl.BlockSpec((D, D), full),
            pl.BlockSpec((1, D), full),
            pl.BlockSpec((1, D), full),
            pl.BlockSpec((D, D), full),
            pl.BlockSpec((1, D), full),
            pl.BlockSpec((1, D), full),
        ],
        out_specs=pl.BlockSpec((512, D), row),
        out_shape=jax.ShapeDtypeStruct((NP, D), jnp.float32),
    )(agts, a0, a1, awt, nw, nb, lwt, lgw, lgb)


def kernel(actors, actor_idcs, actor_ctrs, nodes, node_idcs, node_ctrs,
           params):
    f32 = jnp.float32
    agt_ctrs = actor_ctrs.reshape(-1, 2)
    ctx_ctrs = node_ctrs.reshape(-1, 2)

    actr2 = jnp.full((NP, D), 9.0, f32).at[:NA, :2].set(agt_ctrs)
    actr2 = actr2.at[:NA, 2:].set(0.0)
    cctr2 = jnp.full((8, NP), 9.0, f32).at[:2, :NC].set(ctx_ctrs.T)
    cctr2 = cctr2.at[2:, :].set(0.0)

    mask = _mask_call(actr2, cctr2)
    flat = jnp.nonzero(mask.reshape(-1), size=ECAP,
                       fill_value=NP * NP)[0].astype(jnp.int32)
    valid = flat < NP * NP
    ei = flat // NP
    ej = flat - ei * NP
    ei2 = jnp.where(valid, ei, 0).reshape(ECAP // GCHUNK, GCHUNK)
    ej2 = jnp.where(valid, ej, 0).reshape(ECAP // GCHUNK, GCHUNK)
    sidx2 = jnp.where(valid, ei, NP - 1).reshape(ECAP // SCHUNK, SCHUNK)

    agts = jnp.zeros((NP, D), f32).at[:NA].set(actors)
    nodes_p = jnp.zeros((NP, D), f32).at[:NC].set(nodes)
    nctr2 = jnp.zeros((NP, D), f32).at[:NC, :2].set(ctx_ctrs)
    zrows = jnp.zeros((ROWS, D), f32)

    for i in range(2):
        p = {k: v[i] for k, v in params.items()}
        w1p = jnp.zeros((D, D), f32).at[:, :2].set(p['dist_w1'])
        ua0, ua1, vc0, vc1 = _pre_call(
            agts, actr2, nodes_p, nctr2,
            p['query_w'].T, p['query_gnw'][None], p['query_gnb'][None],
            w1p.T, p['ctx_w1'][:, D:2 * D].T, p['ctx_w1'][:, 2 * D:].T,
            p['dist_b1'][None])
        g = _sc_gather(ua0, ua1, vc0, vc1, ei2, ej2)
        oute = _edge_call(
            g, p['dist_w2'].T, p['ctx_w1'][:, :D].T, p['ctx_w2'].T,
            p['dist_gnw'][None], p['dist_gnb'][None],
            p['ctx_gnw'][None], p['ctx_gnb'][None])
        acc = _sc_scatter(oute, sidx2, zrows)
        agts = _post_call(
            agts, acc[0], acc[1], p['agt_w'].T, p['norm_w'][None],
            p['norm_b'][None], p['lin_w'].T, p['lin_gnw'][None],
            p['lin_gnb'][None])
    return agts[:NA]
